# baseline (speedup 1.0000x reference)
import jax
import jax.numpy as jnp
from jax.experimental import pallas as pl
from jax.experimental.pallas import tpu as pltpu

T = 128
D = 1024
E = 64
K = 8
F = 512
FS = 2048
FS_CHUNK = 512
NS = FS // FS_CHUNK


def _body(x_ref, wr_ref, wsg_ref, wsu_ref, wsd_ref, wseg_ref,
          wg_ref, wu_ref, wd_ref, out_ref, comb_ref):
    s = pl.program_id(0)
    x = x_ref[:]

    @pl.when(s == 0)
    def _():
        out_ref[:] = jnp.zeros_like(out_ref)
        logits = jax.lax.dot_general(
            x, wr_ref[:], (((1,), (1,)), ((), ())),
            preferred_element_type=jnp.float32)
        m = jnp.max(logits, axis=-1, keepdims=True)
        ex = jnp.exp(logits - m)
        probs = ex / jnp.sum(ex, axis=-1, keepdims=True)
        p = probs
        mask = jnp.zeros((T, E), dtype=jnp.bool_)
        lane = jax.lax.broadcasted_iota(jnp.int32, (T, E), 1)
        for _ in range(K):
            idx = jnp.argmax(p, axis=-1)
            onehot = lane == idx[:, None]
            mask = jnp.logical_or(mask, onehot)
            p = jnp.where(onehot, -jnp.inf, p)
        sel = jnp.where(mask, probs, 0.0)
        comb_ref[:] = sel / jnp.sum(sel, axis=-1, keepdims=True)

    @pl.when(s < NS)
    def _():
        g = jnp.dot(x, wsg_ref[:], preferred_element_type=jnp.float32)
        u = jnp.dot(x, wsu_ref[:], preferred_element_type=jnp.float32)
        h = (g * jax.nn.sigmoid(g)) * u
        out_ref[:] += jnp.dot(h, wsd_ref[:], preferred_element_type=jnp.float32)

        @pl.when(s == NS - 1)
        def _():
            seg = jax.nn.sigmoid(jax.lax.dot_general(
                x, wseg_ref[:], (((1,), (1,)), ((), ())),
                preferred_element_type=jnp.float32))
            out_ref[:] = seg * out_ref[:]

    @pl.when(s >= NS)
    def _():
        e = s - NS
        g = jnp.dot(x, wg_ref[0], preferred_element_type=jnp.float32)
        u = jnp.dot(x, wu_ref[0], preferred_element_type=jnp.float32)
        h = (g * jax.nn.sigmoid(g)) * u
        y = jnp.dot(h, wd_ref[0], preferred_element_type=jnp.float32)
        lane = jax.lax.broadcasted_iota(jnp.int32, (T, E), 1)
        scale = jnp.sum(jnp.where(lane == e, comb_ref[:], 0.0),
                        axis=1, keepdims=True)
        out_ref[:] += scale * y


def kernel(hidden_states, w_router, w_gate, w_up, w_down,
           w_shared_gate_proj, w_shared_up_proj, w_shared_down_proj,
           w_shared_expert_gate):
    x = hidden_states.reshape(T, D)

    def _shared_idx(s):
        return jnp.minimum(s, NS - 1)

    def _expert_idx(s):
        return jnp.maximum(s - NS, 0)

    out = pl.pallas_call(
        _body,
        grid=(NS + E,),
        in_specs=[
            pl.BlockSpec((T, D), lambda s: (0, 0)),
            pl.BlockSpec((E, D), lambda s: (0, 0)),
            pl.BlockSpec((D, FS_CHUNK), lambda s: (0, _shared_idx(s))),
            pl.BlockSpec((D, FS_CHUNK), lambda s: (0, _shared_idx(s))),
            pl.BlockSpec((FS_CHUNK, D), lambda s: (_shared_idx(s), 0)),
            pl.BlockSpec((1, D), lambda s: (0, 0)),
            pl.BlockSpec((1, D, F), lambda s: (_expert_idx(s), 0, 0)),
            pl.BlockSpec((1, D, F), lambda s: (_expert_idx(s), 0, 0)),
            pl.BlockSpec((1, F, D), lambda s: (_expert_idx(s), 0, 0)),
        ],
        out_specs=pl.BlockSpec((T, D), lambda s: (0, 0)),
        out_shape=jax.ShapeDtypeStruct((T, D), jnp.float32),
        scratch_shapes=[pltpu.VMEM((T, E), jnp.float32)],
    )(x, w_router, w_shared_gate_proj, w_shared_up_proj,
      w_shared_down_proj, w_shared_expert_gate, w_gate, w_up, w_down)

    return out


# speedup vs baseline: 1.0492x; 1.0492x over previous
import jax
import jax.numpy as jnp
from jax.experimental import pallas as pl
from jax.experimental.pallas import tpu as pltpu

T = 128
D = 1024
E = 64
K = 8
F = 512
FS = 2048
FS_CHUNK = 512
EPB = 2


def _router_body(x_ref, wr_ref, comb_ref):
    x = x_ref[:]
    logits = jax.lax.dot_general(
        x, wr_ref[:], (((1,), (1,)), ((), ())),
        preferred_element_type=jnp.float32)
    m = jnp.max(logits, axis=-1, keepdims=True)
    ex = jnp.exp(logits - m)
    probs = ex / jnp.sum(ex, axis=-1, keepdims=True)
    p = probs
    mask = jnp.zeros((T, E), dtype=jnp.bool_)
    lane = jax.lax.broadcasted_iota(jnp.int32, (T, E), 1)
    for _ in range(K):
        idx = jnp.argmax(p, axis=-1)
        onehot = lane == idx[:, None]
        mask = jnp.logical_or(mask, onehot)
        p = jnp.where(onehot, -jnp.inf, p)
    sel = jnp.where(mask, probs, 0.0)
    comb_ref[:] = sel / jnp.sum(sel, axis=-1, keepdims=True)


def _shared_body(x_ref, wsg_ref, wsu_ref, wsd_ref, wseg_ref, out_ref):
    c = pl.program_id(0)
    x = x_ref[:]
    g = jnp.dot(x, wsg_ref[:], preferred_element_type=jnp.float32)
    u = jnp.dot(x, wsu_ref[:], preferred_element_type=jnp.float32)
    h = (g * jax.nn.sigmoid(g)) * u
    y = jnp.dot(h, wsd_ref[:], preferred_element_type=jnp.float32)

    @pl.when(c == 0)
    def _():
        out_ref[:] = jnp.zeros_like(out_ref)

    out_ref[:] += y

    @pl.when(c == (FS // FS_CHUNK) - 1)
    def _():
        seg = jax.nn.sigmoid(jax.lax.dot_general(
            x, wseg_ref[:], (((1,), (1,)), ((), ())),
            preferred_element_type=jnp.float32))
        out_ref[:] = seg * out_ref[:]


def _expert_body(x_ref, comb_ref, shared_ref, wg_ref, wu_ref, wd_ref, out_ref):
    b = pl.program_id(0)
    x = x_ref[:]
    lane = jax.lax.broadcasted_iota(jnp.int32, (T, E), 1)

    @pl.when(b == 0)
    def _():
        out_ref[:] = shared_ref[:]

    acc = jnp.zeros((T, D), jnp.float32)
    for j in range(EPB):
        g = jnp.dot(x, wg_ref[j], preferred_element_type=jnp.float32)
        u = jnp.dot(x, wu_ref[j], preferred_element_type=jnp.float32)
        h = (g * jax.nn.sigmoid(g)) * u
        y = jnp.dot(h, wd_ref[j], preferred_element_type=jnp.float32)
        scale = jnp.sum(jnp.where(lane == b * EPB + j, comb_ref[:], 0.0),
                        axis=1, keepdims=True)
        acc += scale * y
    out_ref[:] += acc


def kernel(hidden_states, w_router, w_gate, w_up, w_down,
           w_shared_gate_proj, w_shared_up_proj, w_shared_down_proj,
           w_shared_expert_gate):
    x = hidden_states.reshape(T, D)

    comb = pl.pallas_call(
        _router_body,
        out_shape=jax.ShapeDtypeStruct((T, E), jnp.float32),
    )(x, w_router)

    n_chunks = FS // FS_CHUNK
    shared_out = pl.pallas_call(
        _shared_body,
        grid=(n_chunks,),
        in_specs=[
            pl.BlockSpec((T, D), lambda c: (0, 0)),
            pl.BlockSpec((D, FS_CHUNK), lambda c: (0, c)),
            pl.BlockSpec((D, FS_CHUNK), lambda c: (0, c)),
            pl.BlockSpec((FS_CHUNK, D), lambda c: (c, 0)),
            pl.BlockSpec((1, D), lambda c: (0, 0)),
        ],
        out_specs=pl.BlockSpec((T, D), lambda c: (0, 0)),
        out_shape=jax.ShapeDtypeStruct((T, D), jnp.float32),
    )(x, w_shared_gate_proj, w_shared_up_proj, w_shared_down_proj,
      w_shared_expert_gate)

    out = pl.pallas_call(
        _expert_body,
        grid=(E // EPB,),
        in_specs=[
            pl.BlockSpec((T, D), lambda b: (0, 0)),
            pl.BlockSpec((T, E), lambda b: (0, 0)),
            pl.BlockSpec((T, D), lambda b: (0, 0)),
            pl.BlockSpec((EPB, D, F), lambda b: (b, 0, 0)),
            pl.BlockSpec((EPB, D, F), lambda b: (b, 0, 0)),
            pl.BlockSpec((EPB, F, D), lambda b: (b, 0, 0)),
        ],
        out_specs=pl.BlockSpec((T, D), lambda b: (0, 0)),
        out_shape=jax.ShapeDtypeStruct((T, D), jnp.float32),
    )(x, comb, shared_out, w_gate, w_up, w_down)

    return out
